# baseline (speedup 1.0000x reference)
import functools

import jax
import jax.numpy as jnp
from jax import lax
from jax.experimental import pallas as pl
from jax.experimental.pallas import tpu as pltpu
from jax.experimental.pallas import tpu_sc as plsc

BATCH = 4
SEQ = 4096
D_MODEL = 1024
B_TOTAL = BATCH * SEQ

_NC, _NS = 2, 16
_NW = _NC * _NS
B_PER_W = B_TOTAL // _NW
W_PER_ROW = SEQ // B_PER_W
CHUNK = 8
NCHUNK = B_PER_W // CHUNK
NBUF = 12
_LOOP_CHUNKS = ((NCHUNK - NBUF) // NBUF) * NBUF


_mesh = plsc.VectorSubcoreMesh(core_axis_name="c", subcore_axis_name="s")


@functools.partial(
    pl.kernel,
    out_type=jax.ShapeDtypeStruct((B_TOTAL, D_MODEL), jnp.float32),
    mesh=_mesh,
    scratch_types=[
        pltpu.VMEM((B_PER_W,), jnp.int32),
        pltpu.VMEM((NBUF, CHUNK, D_MODEL), jnp.float32),
        [pltpu.SemaphoreType.DMA] * NBUF,
        [pltpu.SemaphoreType.DMA] * NBUF,
    ],
)
def _embed_sc(table_hbm, tok_hbm, out_hbm, idx_v, ring, sems_in, sems_out):
    wid = lax.axis_index("s") * _NC + lax.axis_index("c")
    base = wid * B_PER_W
    row = wid // W_PER_ROW
    col = (wid % W_PER_ROW) * B_PER_W
    pltpu.sync_copy(tok_hbm.at[row, pl.ds(col, B_PER_W)], idx_v)

    def gather(off, b):
        return pltpu.async_copy(
            table_hbm.at[idx_v.at[pl.ds(off, CHUNK)]],
            ring.at[b],
            sems_in[b],
        )

    def writeback(off, b):
        return pltpu.async_copy(
            ring.at[b], out_hbm.at[pl.ds(base + off, CHUNK)], sems_out[b]
        )

    def wait_gather(b):
        pltpu.make_async_copy(
            table_hbm.at[pl.ds(0, CHUNK)], ring.at[b], sems_in[b]
        ).wait()

    def wait_writeback(b):
        pltpu.make_async_copy(
            ring.at[b], out_hbm.at[pl.ds(base, CHUNK)], sems_out[b]
        ).wait()

    for b in range(NBUF):
        gather(b * CHUNK, b)

    @pl.loop(0, _LOOP_CHUNKS, step=NBUF)
    def _steady(g):
        goff = g * CHUNK
        for b in range(NBUF):
            off = goff + b * CHUNK
            wait_gather(b)
            writeback(off, b)
            wait_writeback(b)
            gather(off + NBUF * CHUNK, b)

    for c in range(_LOOP_CHUNKS, NCHUNK):
        b = c % NBUF
        off = c * CHUNK
        wait_gather(b)
        writeback(off, b)
        nxt = c + NBUF
        if nxt < NCHUNK:
            wait_writeback(b)
            gather(nxt * CHUNK, b)
    for c in range(NCHUNK - NBUF, NCHUNK):
        wait_writeback(c % NBUF)


def kernel(tokens, W_E):
    out = _embed_sc(W_E, tokens.astype(jnp.int32))
    return out.reshape(tokens.shape + (W_E.shape[1],))


# speedup vs baseline: 1.0051x; 1.0051x over previous
import functools

import jax
import jax.numpy as jnp
from jax import lax
from jax.experimental import pallas as pl
from jax.experimental.pallas import tpu as pltpu
from jax.experimental.pallas import tpu_sc as plsc

BATCH = 4
SEQ = 4096
D_MODEL = 1024
B_TOTAL = BATCH * SEQ

_NC, _NS = 2, 16
_NW = _NC * _NS
B_PER_W = B_TOTAL // _NW
W_PER_ROW = SEQ // B_PER_W
CHUNK = 16
NCHUNK = B_PER_W // CHUNK
NBUF = 7
_LOOP_CHUNKS = ((NCHUNK - NBUF) // NBUF) * NBUF


_mesh = plsc.VectorSubcoreMesh(core_axis_name="c", subcore_axis_name="s")


@functools.partial(
    pl.kernel,
    out_type=jax.ShapeDtypeStruct((B_TOTAL, D_MODEL), jnp.float32),
    mesh=_mesh,
    scratch_types=[
        pltpu.VMEM((B_PER_W,), jnp.int32),
        pltpu.VMEM((NBUF, CHUNK, D_MODEL), jnp.float32),
        [pltpu.SemaphoreType.DMA] * NBUF,
        [pltpu.SemaphoreType.DMA] * NBUF,
    ],
)
def _embed_sc(table_hbm, tok_hbm, out_hbm, idx_v, ring, sems_in, sems_out):
    wid = lax.axis_index("s") * _NC + lax.axis_index("c")
    base = wid * B_PER_W
    row = wid // W_PER_ROW
    col = (wid % W_PER_ROW) * B_PER_W
    pltpu.sync_copy(tok_hbm.at[row, pl.ds(col, B_PER_W)], idx_v)

    def gather(off, b):
        return pltpu.async_copy(
            table_hbm.at[idx_v.at[pl.ds(off, CHUNK)]],
            ring.at[b],
            sems_in[b],
        )

    def writeback(off, b):
        return pltpu.async_copy(
            ring.at[b], out_hbm.at[pl.ds(base + off, CHUNK)], sems_out[b]
        )

    def wait_gather(b):
        pltpu.make_async_copy(
            table_hbm.at[pl.ds(0, CHUNK)], ring.at[b], sems_in[b]
        ).wait()

    def wait_writeback(b):
        pltpu.make_async_copy(
            ring.at[b], out_hbm.at[pl.ds(base, CHUNK)], sems_out[b]
        ).wait()

    for b in range(NBUF):
        gather(b * CHUNK, b)

    @pl.loop(0, _LOOP_CHUNKS, step=NBUF)
    def _steady(g):
        goff = g * CHUNK
        for b in range(NBUF):
            off = goff + b * CHUNK
            wait_gather(b)
            writeback(off, b)
            wait_writeback(b)
            gather(off + NBUF * CHUNK, b)

    for c in range(_LOOP_CHUNKS, NCHUNK):
        b = c % NBUF
        off = c * CHUNK
        wait_gather(b)
        writeback(off, b)
        nxt = c + NBUF
        if nxt < NCHUNK:
            wait_writeback(b)
            gather(nxt * CHUNK, b)
    for c in range(NCHUNK - NBUF, NCHUNK):
        wait_writeback(c % NBUF)


def kernel(tokens, W_E):
    out = _embed_sc(W_E, tokens.astype(jnp.int32))
    return out.reshape(tokens.shape + (W_E.shape[1],))


# speedup vs baseline: 1.0155x; 1.0104x over previous
import functools

import jax
import jax.numpy as jnp
from jax import lax
from jax.experimental import pallas as pl
from jax.experimental.pallas import tpu as pltpu
from jax.experimental.pallas import tpu_sc as plsc

BATCH = 4
SEQ = 4096
D_MODEL = 1024
B_TOTAL = BATCH * SEQ

_NC, _NS = 2, 16
_NW = _NC * _NS
B_PER_W = B_TOTAL // _NW
W_PER_ROW = SEQ // B_PER_W
CHUNK = 16
NCHUNK = B_PER_W // CHUNK
NBUF = 6
_LOOP_CHUNKS = ((NCHUNK - NBUF) // NBUF) * NBUF


_mesh = plsc.VectorSubcoreMesh(core_axis_name="c", subcore_axis_name="s")


@functools.partial(
    pl.kernel,
    out_type=jax.ShapeDtypeStruct((B_TOTAL, D_MODEL), jnp.float32),
    mesh=_mesh,
    scratch_types=[
        pltpu.VMEM((B_PER_W,), jnp.int32),
        pltpu.VMEM((NBUF, CHUNK, D_MODEL), jnp.float32),
        [pltpu.SemaphoreType.DMA] * NBUF,
        [pltpu.SemaphoreType.DMA] * NBUF,
    ],
)
def _embed_sc(table_hbm, tok_hbm, out_hbm, idx_v, ring, sems_in, sems_out):
    wid = lax.axis_index("s") * _NC + lax.axis_index("c")
    base = wid * B_PER_W
    row = wid // W_PER_ROW
    col = (wid % W_PER_ROW) * B_PER_W
    pltpu.sync_copy(tok_hbm.at[row, pl.ds(col, B_PER_W)], idx_v)

    def gather(off, b):
        return pltpu.async_copy(
            table_hbm.at[idx_v.at[pl.ds(off, CHUNK)]],
            ring.at[b],
            sems_in[b],
        )

    def writeback(off, b):
        return pltpu.async_copy(
            ring.at[b], out_hbm.at[pl.ds(base + off, CHUNK)], sems_out[b]
        )

    def wait_gather(b):
        pltpu.make_async_copy(
            table_hbm.at[pl.ds(0, CHUNK)], ring.at[b], sems_in[b]
        ).wait()

    def wait_writeback(b):
        pltpu.make_async_copy(
            ring.at[b], out_hbm.at[pl.ds(base, CHUNK)], sems_out[b]
        ).wait()

    for b in range(NBUF):
        gather(b * CHUNK, b)

    @pl.loop(0, _LOOP_CHUNKS, step=NBUF)
    def _steady(g):
        goff = g * CHUNK
        for b in range(NBUF):
            off = goff + b * CHUNK
            wait_gather(b)
            writeback(off, b)
            wait_writeback(b)
            gather(off + NBUF * CHUNK, b)

    for c in range(_LOOP_CHUNKS, NCHUNK):
        b = c % NBUF
        off = c * CHUNK
        wait_gather(b)
        writeback(off, b)
        nxt = c + NBUF
        if nxt < NCHUNK:
            wait_writeback(b)
            gather(nxt * CHUNK, b)
    for c in range(NCHUNK - NBUF, NCHUNK):
        wait_writeback(c % NBUF)


def kernel(tokens, W_E):
    out = _embed_sc(W_E, tokens.astype(jnp.int32))
    return out.reshape(tokens.shape + (W_E.shape[1],))
